# Initial kernel scaffold; baseline (speedup 1.0000x reference)
import jax
import jax.numpy as jnp
from jax.experimental import pallas as pl


def kernel(h, edge_index, proj_W, proj_b, proj_ln_g, proj_ln_b, mp_W, mp_b, mp_ln_g, mp_ln_b, W1, b1, ln_g, ln_b, W2, b2):
    raise NotImplementedError("write your pallas kernel here")


# speedup vs baseline: 3.5967x; 3.5967x over previous
import functools

import jax
import jax.numpy as jnp
from jax import lax
from jax.experimental import pallas as pl
from jax.experimental.pallas import tpu as pltpu
from jax.experimental.pallas import tpu_sc as plsc

N = 10000
E = 160000
F = 256
FH = 128
CLASSES = 2
DEGW = 16

NSUB = 16
NCORE = 2
NPAD = 10240
ROWS_PER_SUB = NPAD // NSUB
ZROWS = 64
AGG_CHUNK = 200
EDGE_CHUNK = 200

RN = 1024
RE = 2000


def _ln(y, g, b, eps=1e-5):
    m = jnp.mean(y, axis=-1, keepdims=True)
    v = jnp.mean((y - m) ** 2, axis=-1, keepdims=True)
    return (y - m) * lax.rsqrt(v + eps) * g + b


def _proj_body(h_ref, W_ref, b_ref, g_ref, bb_ref, xlo_ref, xhi_ref):
    y0 = jnp.dot(h_ref[:, :FH], W_ref[0], preferred_element_type=jnp.float32) + b_ref[0]
    y1 = jnp.dot(h_ref[:, FH:], W_ref[1], preferred_element_type=jnp.float32) + b_ref[1]
    xlo_ref[...] = jax.nn.relu(_ln(y0, g_ref[0], bb_ref[0]))
    xhi_ref[...] = jax.nn.relu(_ln(y1, g_ref[1], bb_ref[1]))


def _proj(h, proj_W, proj_b, proj_ln_g, proj_ln_b):
    return pl.pallas_call(
        _proj_body,
        grid=(NPAD // RN,),
        in_specs=[
            pl.BlockSpec((RN, F), lambda i: (i, 0)),
            pl.BlockSpec((2, FH, FH), lambda i: (0, 0, 0)),
            pl.BlockSpec((2, FH), lambda i: (0, 0)),
            pl.BlockSpec((2, FH), lambda i: (0, 0)),
            pl.BlockSpec((2, FH), lambda i: (0, 0)),
        ],
        out_specs=[pl.BlockSpec((RN, FH), lambda i: (i, 0))] * 2,
        out_shape=[jax.ShapeDtypeStruct((NPAD, FH), jnp.float32)] * 2,
    )(h, proj_W, proj_b, proj_ln_g, proj_ln_b)


def _layer_common(xlo, xhi, alo, ahi, deg0_ref, deg1_ref, W, b):
    i = pl.program_id(0)
    d = deg0_ref[pl.ds(i * RN, RN)] + deg1_ref[pl.ds(i * RN, RN)]
    d = d.reshape(-1, 1)
    norm = jnp.where(d > 0, 1.0 / d, 0.0)
    y = (jnp.dot(xlo, W[:FH], preferred_element_type=jnp.float32)
         + jnp.dot(xhi, W[FH:F], preferred_element_type=jnp.float32)
         + jnp.dot(alo * norm, W[F:F + FH], preferred_element_type=jnp.float32)
         + jnp.dot(ahi * norm, W[F + FH:], preferred_element_type=jnp.float32)
         + b)
    return y


def _layer_body(xlo_ref, xhi_ref, alo_ref, ahi_ref, deg0_ref, deg1_ref, W_ref, b_ref,
                g_ref, bb_ref, ylo_ref, yhi_ref):
    y = _layer_common(xlo_ref[...], xhi_ref[...], alo_ref[...], ahi_ref[...],
                      deg0_ref, deg1_ref, W_ref[...], b_ref[...])
    y = jax.nn.relu(_ln(y, g_ref[...], bb_ref[...]))
    ylo_ref[...] = y[:, :FH]
    yhi_ref[...] = y[:, FH:]


def _layer(xlo, xhi, alo, ahi, deg0, deg1, W, b, g, bb):
    return pl.pallas_call(
        _layer_body,
        grid=(NPAD // RN,),
        in_specs=[
            pl.BlockSpec((RN, FH), lambda i: (i, 0)),
            pl.BlockSpec((RN, FH), lambda i: (i, 0)),
            pl.BlockSpec((RN, FH), lambda i: (i, 0)),
            pl.BlockSpec((RN, FH), lambda i: (i, 0)),
            pl.BlockSpec((NPAD,), lambda i: (0,)),
            pl.BlockSpec((NPAD,), lambda i: (0,)),
            pl.BlockSpec((2 * F, F), lambda i: (0, 0)),
            pl.BlockSpec((1, F), lambda i: (0, 0)),
            pl.BlockSpec((1, F), lambda i: (0, 0)),
            pl.BlockSpec((1, F), lambda i: (0, 0)),
        ],
        out_specs=[pl.BlockSpec((RN, FH), lambda i: (i, 0))] * 2,
        out_shape=[jax.ShapeDtypeStruct((NPAD, FH), jnp.float32)] * 2,
    )(xlo, xhi, alo, ahi, deg0, deg1, W, b, g, bb)


def _layer_ab_body(xlo_ref, xhi_ref, alo_ref, ahi_ref, deg0_ref, deg1_ref, W_ref, b_ref,
                   g_ref, bb_ref, W1_ref, b1_ref, A_ref, B_ref):
    y = _layer_common(xlo_ref[...], xhi_ref[...], alo_ref[...], ahi_ref[...],
                      deg0_ref, deg1_ref, W_ref[...], b_ref[...])
    y = jax.nn.relu(_ln(y, g_ref[...], bb_ref[...]))
    A_ref[...] = jnp.dot(y, W1_ref[:F], preferred_element_type=jnp.float32)
    B_ref[...] = jnp.dot(y, W1_ref[F:], preferred_element_type=jnp.float32) + b1_ref[...]


def _layer_ab(xlo, xhi, alo, ahi, deg0, deg1, W, b, g, bb, W1, b1):
    return pl.pallas_call(
        _layer_ab_body,
        grid=(NPAD // RN,),
        in_specs=[
            pl.BlockSpec((RN, FH), lambda i: (i, 0)),
            pl.BlockSpec((RN, FH), lambda i: (i, 0)),
            pl.BlockSpec((RN, FH), lambda i: (i, 0)),
            pl.BlockSpec((RN, FH), lambda i: (i, 0)),
            pl.BlockSpec((NPAD,), lambda i: (0,)),
            pl.BlockSpec((NPAD,), lambda i: (0,)),
            pl.BlockSpec((2 * F, F), lambda i: (0, 0)),
            pl.BlockSpec((1, F), lambda i: (0, 0)),
            pl.BlockSpec((1, F), lambda i: (0, 0)),
            pl.BlockSpec((1, F), lambda i: (0, 0)),
            pl.BlockSpec((2 * F, F), lambda i: (0, 0)),
            pl.BlockSpec((1, F), lambda i: (0, 0)),
        ],
        out_specs=[pl.BlockSpec((RN, F), lambda i: (i, 0))] * 2,
        out_shape=[jax.ShapeDtypeStruct((NPAD, F), jnp.float32)] * 2,
    )(xlo, xhi, alo, ahi, deg0, deg1, W, b, g, bb, W1, b1)


def _final_body(e_ref, g_ref, b_ref, W2_ref, b2_ref, o_ref):
    e = jax.nn.relu(_ln(e_ref[...], g_ref[...], b_ref[...]))
    o_ref[...] = jnp.dot(e, W2_ref[...], preferred_element_type=jnp.float32) + b2_ref[...]


def _final(e_pre, g, b, W2, b2):
    return pl.pallas_call(
        _final_body,
        grid=(E // RE,),
        in_specs=[
            pl.BlockSpec((RE, F), lambda i: (i, 0)),
            pl.BlockSpec((1, F), lambda i: (0, 0)),
            pl.BlockSpec((1, F), lambda i: (0, 0)),
            pl.BlockSpec((F, CLASSES), lambda i: (0, 0)),
            pl.BlockSpec((1, CLASSES), lambda i: (0, 0)),
        ],
        out_specs=pl.BlockSpec((RE, CLASSES), lambda i: (i, 0)),
        out_shape=jax.ShapeDtypeStruct((E, CLASSES), jnp.float32),
    )(e_pre, g, b, W2, b2)


@functools.lru_cache(maxsize=None)
def _mesh():
    return plsc.VectorSubcoreMesh(core_axis_name="c", subcore_axis_name="s")


@functools.lru_cache(maxsize=None)
def _make_agg():
    def body(xlo, xhi, src, dst, agglo, agghi, acc, srcv, dstv, rows, zbuf, sem):
        cid = lax.axis_index("c")
        sid = lax.axis_index("s")
        r0 = sid * ROWS_PER_SUB

        z16 = jnp.zeros((16,), jnp.float32)

        def zb(i, c):
            zbuf[i // 8, pl.ds((i % 8) * 16, 16)] = z16
            return c
        lax.fori_loop(0, ZROWS * 8, zb, 0)

        for j in range(ROWS_PER_SUB // ZROWS):
            pltpu.sync_copy(zbuf, acc.at[pl.ds(r0 + j * ZROWS, ZROWS)])

        plsc.subcore_barrier()

        eper = E // NSUB
        def chunk(i, c):
            b = sid * eper + i * AGG_CHUNK
            pltpu.sync_copy(src.at[pl.ds(b, AGG_CHUNK)], srcv)
            pltpu.sync_copy(dst.at[pl.ds(b, AGG_CHUNK)], dstv)

            @pl.when(cid == 0)
            def _():
                pltpu.async_copy(xlo.at[srcv], rows, sem).wait()

            @pl.when(cid == 1)
            def _():
                pltpu.async_copy(xhi.at[srcv], rows, sem).wait()

            pltpu.sync_copy(rows, acc.at[dstv], add=True)
            return c
        lax.fori_loop(0, eper // AGG_CHUNK, chunk, 0)

        plsc.subcore_barrier()

        for j in range(ROWS_PER_SUB // ZROWS):
            @pl.when(cid == 0)
            def _():
                pltpu.sync_copy(acc.at[pl.ds(r0 + j * ZROWS, ZROWS)], zbuf)
                pltpu.sync_copy(zbuf, agglo.at[pl.ds(r0 + j * ZROWS, ZROWS)])

            @pl.when(cid == 1)
            def _():
                pltpu.sync_copy(acc.at[pl.ds(r0 + j * ZROWS, ZROWS)], zbuf)
                pltpu.sync_copy(zbuf, agghi.at[pl.ds(r0 + j * ZROWS, ZROWS)])

    return pl.kernel(
        body,
        out_type=(jax.ShapeDtypeStruct((NPAD, FH), jnp.float32),
                  jax.ShapeDtypeStruct((NPAD, FH), jnp.float32)),
        mesh=_mesh(),
        scratch_types=[
            pltpu.VMEM_SHARED((NPAD, FH), jnp.float32),
            pltpu.VMEM((AGG_CHUNK,), jnp.int32),
            pltpu.VMEM((AGG_CHUNK,), jnp.int32),
            pltpu.VMEM((AGG_CHUNK, FH), jnp.float32),
            pltpu.VMEM((ZROWS, FH), jnp.float32),
            pltpu.SemaphoreType.DMA,
        ],
    )


DEG_CHUNK = 1000


@functools.lru_cache(maxsize=None)
def _make_deg():
    def body(dst, deg0, deg1, dacc, dstv, obuf, sem):
        cid = lax.axis_index("c")
        sid = lax.axis_index("s")
        r0 = sid * ROWS_PER_SUB

        z16 = jnp.zeros((16,), jnp.float32)
        o16 = jnp.ones((16,), jnp.float32)

        def zb(i, c):
            obuf[pl.ds(i * 16, 16)] = z16
            return c
        lax.fori_loop(0, ROWS_PER_SUB // 16, zb, 0)
        pltpu.sync_copy(obuf.at[pl.ds(0, ROWS_PER_SUB)], dacc.at[pl.ds(r0, ROWS_PER_SUB)])

        def ob(i, c):
            obuf[pl.ds(i * 16, 16)] = o16
            return c
        lax.fori_loop(0, (DEG_CHUNK + 15) // 16, ob, 0)

        plsc.subcore_barrier()

        eper = E // (NSUB * NCORE)
        wid = sid * NCORE + cid
        def chunk(i, c):
            b = wid * eper + i * DEG_CHUNK
            pltpu.sync_copy(dst.at[pl.ds(b, DEG_CHUNK)], dstv)
            pltpu.sync_copy(obuf.at[pl.ds(0, DEG_CHUNK)], dacc.at[dstv], add=True)
            return c
        lax.fori_loop(0, eper // DEG_CHUNK, chunk, 0)

        plsc.subcore_barrier()

        pltpu.sync_copy(dacc.at[pl.ds(r0, ROWS_PER_SUB)], obuf.at[pl.ds(0, ROWS_PER_SUB)])

        @pl.when(cid == 0)
        def _():
            pltpu.sync_copy(obuf.at[pl.ds(0, ROWS_PER_SUB)], deg0.at[pl.ds(r0, ROWS_PER_SUB)])

        @pl.when(cid == 1)
        def _():
            pltpu.sync_copy(obuf.at[pl.ds(0, ROWS_PER_SUB)], deg1.at[pl.ds(r0, ROWS_PER_SUB)])

    return pl.kernel(
        body,
        out_type=(jax.ShapeDtypeStruct((NPAD,), jnp.float32),
                  jax.ShapeDtypeStruct((NPAD,), jnp.float32)),
        mesh=_mesh(),
        scratch_types=[
            pltpu.VMEM_SHARED((NPAD,), jnp.float32),
            pltpu.VMEM((DEG_CHUNK,), jnp.int32),
            pltpu.VMEM((((DEG_CHUNK + 15) // 16) * 16,), jnp.float32),
            pltpu.SemaphoreType.DMA,
        ],
    )


def _edge_body(A, B, src, dst, out, srcv, dstv, arows, brows, sem):
    cid = lax.axis_index("c")
    sid = lax.axis_index("s")
    wid = sid * NCORE + cid
    eper = E // (NSUB * NCORE)

    def chunk(i, c):
        b = wid * eper + i * EDGE_CHUNK
        pltpu.sync_copy(src.at[pl.ds(b, EDGE_CHUNK)], srcv)
        pltpu.sync_copy(dst.at[pl.ds(b, EDGE_CHUNK)], dstv)
        pltpu.async_copy(A.at[srcv], arows, sem).wait()
        pltpu.async_copy(B.at[dstv], brows, sem).wait()

        def addrow(r, c2):
            for cc in range(F // 16):
                s = pl.ds(cc * 16, 16)
                arows[r, s] = arows[r, s] + brows[r, s]
            return c2
        lax.fori_loop(0, EDGE_CHUNK, addrow, 0)
        pltpu.sync_copy(arows, out.at[pl.ds(b, EDGE_CHUNK)])
        return c
    lax.fori_loop(0, eper // EDGE_CHUNK, chunk, 0)


@functools.lru_cache(maxsize=None)
def _make_edge():
    return pl.kernel(
        _edge_body,
        out_type=jax.ShapeDtypeStruct((E, F), jnp.float32),
        mesh=_mesh(),
        scratch_types=[
            pltpu.VMEM((EDGE_CHUNK,), jnp.int32),
            pltpu.VMEM((EDGE_CHUNK,), jnp.int32),
            pltpu.VMEM((EDGE_CHUNK, F), jnp.float32),
            pltpu.VMEM((EDGE_CHUNK, F), jnp.float32),
            pltpu.SemaphoreType.DMA,
        ],
    )


def kernel(h, edge_index, proj_W, proj_b, proj_ln_g, proj_ln_b,
           mp_W, mp_b, mp_ln_g, mp_ln_b, W1, b1, ln_g, ln_b, W2, b2):
    src = edge_index[0]
    dst = edge_index[1]

    deg0, deg1 = _make_deg()(dst)
    h_pad = jnp.pad(h, ((0, NPAD - N), (0, 0)))
    xlo, xhi = _proj(h_pad, proj_W, proj_b, proj_ln_g, proj_ln_b)
    agglo, agghi = _make_agg()(xlo, xhi, src, dst)
    ylo, yhi = _layer(xlo, xhi, agglo, agghi, deg0, deg1,
                      mp_W[0], mp_b[0].reshape(1, F),
                      mp_ln_g[0].reshape(1, F), mp_ln_b[0].reshape(1, F))
    agglo2, agghi2 = _make_agg()(ylo, yhi, src, dst)
    A, Bm = _layer_ab(ylo, yhi, agglo2, agghi2, deg0, deg1,
                      mp_W[1], mp_b[1].reshape(1, F),
                      mp_ln_g[1].reshape(1, F), mp_ln_b[1].reshape(1, F),
                      W1, b1.reshape(1, F))
    e_pre = _make_edge()(A, Bm, src, dst)
    return _final(e_pre, ln_g.reshape(1, F), ln_b.reshape(1, F), W2, b2.reshape(1, CLASSES))


# speedup vs baseline: 4.9996x; 1.3901x over previous
import functools

import jax
import jax.numpy as jnp
from jax import lax
from jax.experimental import pallas as pl
from jax.experimental.pallas import tpu as pltpu
from jax.experimental.pallas import tpu_sc as plsc

N = 10000
E = 160000
F = 256
FH = 128
CLASSES = 2
DEGW = 16

NSUB = 16
NCORE = 2
NPAD = 10240
ROWS_PER_SUB = NPAD // NSUB
AGG_CHUNK = 80
AGG_NCH = E // (NSUB * AGG_CHUNK)
EDGE_CHUNK = 40
EDGE_NCH = E // (NSUB * NCORE * EDGE_CHUNK)

RN = 1024
RE = 2000


def _ln(y, g, b, eps=1e-5):
    m = jnp.mean(y, axis=-1, keepdims=True)
    v = jnp.mean((y - m) ** 2, axis=-1, keepdims=True)
    return (y - m) * lax.rsqrt(v + eps) * g + b


def _proj_body(h_ref, W_ref, b_ref, g_ref, bb_ref, xlo_ref, xhi_ref):
    y0 = jnp.dot(h_ref[:, :FH], W_ref[0], preferred_element_type=jnp.float32) + b_ref[0]
    y1 = jnp.dot(h_ref[:, FH:], W_ref[1], preferred_element_type=jnp.float32) + b_ref[1]
    xlo_ref[...] = jax.nn.relu(_ln(y0, g_ref[0], bb_ref[0]))
    xhi_ref[...] = jax.nn.relu(_ln(y1, g_ref[1], bb_ref[1]))


def _proj(h, proj_W, proj_b, proj_ln_g, proj_ln_b):
    return pl.pallas_call(
        _proj_body,
        grid=(NPAD // RN,),
        in_specs=[
            pl.BlockSpec((RN, F), lambda i: (i, 0)),
            pl.BlockSpec((2, FH, FH), lambda i: (0, 0, 0)),
            pl.BlockSpec((2, FH), lambda i: (0, 0)),
            pl.BlockSpec((2, FH), lambda i: (0, 0)),
            pl.BlockSpec((2, FH), lambda i: (0, 0)),
        ],
        out_specs=[pl.BlockSpec((RN, FH), lambda i: (i, 0))] * 2,
        out_shape=[jax.ShapeDtypeStruct((NPAD, FH), jnp.float32)] * 2,
    )(h, proj_W, proj_b, proj_ln_g, proj_ln_b)


def _layer_common(xlo, xhi, alo, ahi, deg0_ref, deg1_ref, W, b):
    i = pl.program_id(0)
    d = deg0_ref[pl.ds(i * RN, RN)] + deg1_ref[pl.ds(i * RN, RN)]
    d = d.reshape(-1, 1)
    norm = jnp.where(d > 0, 1.0 / d, 0.0)
    y = (jnp.dot(xlo, W[:FH], preferred_element_type=jnp.float32)
         + jnp.dot(xhi, W[FH:F], preferred_element_type=jnp.float32)
         + jnp.dot(alo * norm, W[F:F + FH], preferred_element_type=jnp.float32)
         + jnp.dot(ahi * norm, W[F + FH:], preferred_element_type=jnp.float32)
         + b)
    return y


def _layer_body(xlo_ref, xhi_ref, alo_ref, ahi_ref, deg0_ref, deg1_ref, W_ref, b_ref,
                g_ref, bb_ref, ylo_ref, yhi_ref):
    y = _layer_common(xlo_ref[...], xhi_ref[...], alo_ref[...], ahi_ref[...],
                      deg0_ref, deg1_ref, W_ref[...], b_ref[...])
    y = jax.nn.relu(_ln(y, g_ref[...], bb_ref[...]))
    ylo_ref[...] = y[:, :FH]
    yhi_ref[...] = y[:, FH:]


def _layer(xlo, xhi, alo, ahi, deg0, deg1, W, b, g, bb):
    return pl.pallas_call(
        _layer_body,
        grid=(NPAD // RN,),
        in_specs=[
            pl.BlockSpec((RN, FH), lambda i: (i, 0)),
            pl.BlockSpec((RN, FH), lambda i: (i, 0)),
            pl.BlockSpec((RN, FH), lambda i: (i, 0)),
            pl.BlockSpec((RN, FH), lambda i: (i, 0)),
            pl.BlockSpec((NPAD,), lambda i: (0,)),
            pl.BlockSpec((NPAD,), lambda i: (0,)),
            pl.BlockSpec((2 * F, F), lambda i: (0, 0)),
            pl.BlockSpec((1, F), lambda i: (0, 0)),
            pl.BlockSpec((1, F), lambda i: (0, 0)),
            pl.BlockSpec((1, F), lambda i: (0, 0)),
        ],
        out_specs=[pl.BlockSpec((RN, FH), lambda i: (i, 0))] * 2,
        out_shape=[jax.ShapeDtypeStruct((NPAD, FH), jnp.float32)] * 2,
    )(xlo, xhi, alo, ahi, deg0, deg1, W, b, g, bb)


def _layer_ab_body(xlo_ref, xhi_ref, alo_ref, ahi_ref, deg0_ref, deg1_ref, W_ref, b_ref,
                   g_ref, bb_ref, W1_ref, b1_ref, A_ref, B_ref):
    y = _layer_common(xlo_ref[...], xhi_ref[...], alo_ref[...], ahi_ref[...],
                      deg0_ref, deg1_ref, W_ref[...], b_ref[...])
    y = jax.nn.relu(_ln(y, g_ref[...], bb_ref[...]))
    A_ref[...] = jnp.dot(y, W1_ref[:F], preferred_element_type=jnp.float32)
    B_ref[...] = jnp.dot(y, W1_ref[F:], preferred_element_type=jnp.float32) + b1_ref[...]


def _layer_ab(xlo, xhi, alo, ahi, deg0, deg1, W, b, g, bb, W1, b1):
    return pl.pallas_call(
        _layer_ab_body,
        grid=(NPAD // RN,),
        in_specs=[
            pl.BlockSpec((RN, FH), lambda i: (i, 0)),
            pl.BlockSpec((RN, FH), lambda i: (i, 0)),
            pl.BlockSpec((RN, FH), lambda i: (i, 0)),
            pl.BlockSpec((RN, FH), lambda i: (i, 0)),
            pl.BlockSpec((NPAD,), lambda i: (0,)),
            pl.BlockSpec((NPAD,), lambda i: (0,)),
            pl.BlockSpec((2 * F, F), lambda i: (0, 0)),
            pl.BlockSpec((1, F), lambda i: (0, 0)),
            pl.BlockSpec((1, F), lambda i: (0, 0)),
            pl.BlockSpec((1, F), lambda i: (0, 0)),
            pl.BlockSpec((2 * F, F), lambda i: (0, 0)),
            pl.BlockSpec((1, F), lambda i: (0, 0)),
        ],
        out_specs=[pl.BlockSpec((RN, F), lambda i: (i, 0))] * 2,
        out_shape=[jax.ShapeDtypeStruct((NPAD, F), jnp.float32)] * 2,
    )(xlo, xhi, alo, ahi, deg0, deg1, W, b, g, bb, W1, b1)


def _final_body(e_ref, g_ref, b_ref, W2_ref, b2_ref, o_ref):
    e = jax.nn.relu(_ln(e_ref[...], g_ref[...], b_ref[...]))
    o_ref[...] = jnp.dot(e, W2_ref[...], preferred_element_type=jnp.float32) + b2_ref[...]


def _final(e_pre, g, b, W2, b2):
    return pl.pallas_call(
        _final_body,
        grid=(E // RE,),
        in_specs=[
            pl.BlockSpec((RE, F), lambda i: (i, 0)),
            pl.BlockSpec((1, F), lambda i: (0, 0)),
            pl.BlockSpec((1, F), lambda i: (0, 0)),
            pl.BlockSpec((F, CLASSES), lambda i: (0, 0)),
            pl.BlockSpec((1, CLASSES), lambda i: (0, 0)),
        ],
        out_specs=pl.BlockSpec((RE, CLASSES), lambda i: (i, 0)),
        out_shape=jax.ShapeDtypeStruct((E, CLASSES), jnp.float32),
    )(e_pre, g, b, W2, b2)


@functools.lru_cache(maxsize=None)
def _mesh():
    return plsc.VectorSubcoreMesh(core_axis_name="c", subcore_axis_name="s")


@functools.lru_cache(maxsize=None)
def _make_agg():
    def body(xlo, xhi, src1, dst3, agglo, agghi, acc, srcv, dstv,
             rows_a, rows_b, sem_a, sem_b):
        cid = lax.axis_index("c")
        sid = lax.axis_index("s")
        r0 = sid * ROWS_PER_SUB

        z16 = jnp.zeros((16,), jnp.float32)

        def zb(i, c):
            rows_a[i // 8, pl.ds((i % 8) * 16, 16)] = z16
            return c
        lax.fori_loop(0, AGG_CHUNK * 8, zb, 0)

        for j in range(ROWS_PER_SUB // AGG_CHUNK):
            pltpu.sync_copy(rows_a, acc.at[pl.ds(r0 + j * AGG_CHUNK, AGG_CHUNK)])

        eper = E // NSUB
        pltpu.sync_copy(src1.at[pl.ds(sid * eper, eper)], srcv)
        pltpu.sync_copy(dst3.at[sid], dstv)

        plsc.subcore_barrier()

        def sidx(i):
            return srcv.at[pl.ds(i * AGG_CHUNK, AGG_CHUNK)]

        def run(xref):
            pltpu.async_copy(xref.at[sidx(0)], rows_a, sem_a)

            def pair(j, c):
                ia = 2 * j
                ib = 2 * j + 1
                pltpu.async_copy(xref.at[sidx(ib)], rows_b, sem_b)
                pltpu.make_async_copy(xref.at[sidx(ia)], rows_a, sem_a).wait()
                pltpu.sync_copy(rows_a, acc.at[dstv.at[ia]], add=True)
                pltpu.async_copy(xref.at[sidx(ib + 1)], rows_a, sem_a)
                pltpu.make_async_copy(xref.at[sidx(ib)], rows_b, sem_b).wait()
                pltpu.sync_copy(rows_b, acc.at[dstv.at[ib]], add=True)
                return c
            lax.fori_loop(0, (AGG_NCH - 1) // 2, pair, 0)

            last = AGG_NCH - 1
            pltpu.make_async_copy(xref.at[sidx(last)], rows_a, sem_a).wait()
            pltpu.sync_copy(rows_a, acc.at[dstv.at[last]], add=True)

        @pl.when(cid == 0)
        def _():
            run(xlo)

        @pl.when(cid == 1)
        def _():
            run(xhi)

        plsc.subcore_barrier()

        for j in range(ROWS_PER_SUB // AGG_CHUNK):
            sl = pl.ds(r0 + j * AGG_CHUNK, AGG_CHUNK)

            @pl.when(cid == 0)
            def _():
                pltpu.sync_copy(acc.at[sl], rows_a)
                pltpu.sync_copy(rows_a, agglo.at[sl])

            @pl.when(cid == 1)
            def _():
                pltpu.sync_copy(acc.at[sl], rows_a)
                pltpu.sync_copy(rows_a, agghi.at[sl])

    return pl.kernel(
        body,
        out_type=(jax.ShapeDtypeStruct((NPAD, FH), jnp.float32),
                  jax.ShapeDtypeStruct((NPAD, FH), jnp.float32)),
        mesh=_mesh(),
        scratch_types=[
            pltpu.VMEM_SHARED((NPAD, FH), jnp.float32),
            pltpu.VMEM((E // NSUB,), jnp.int32),
            pltpu.VMEM((AGG_NCH, AGG_CHUNK), jnp.int32),
            pltpu.VMEM((AGG_CHUNK, FH), jnp.float32),
            pltpu.VMEM((AGG_CHUNK, FH), jnp.float32),
            pltpu.SemaphoreType.DMA,
            pltpu.SemaphoreType.DMA,
        ],
    )


DEG_CHUNK = 1000


@functools.lru_cache(maxsize=None)
def _make_deg():
    def body(dst, deg0, deg1, dacc, dstv, obuf, sem):
        cid = lax.axis_index("c")
        sid = lax.axis_index("s")
        r0 = sid * ROWS_PER_SUB

        z16 = jnp.zeros((16,), jnp.float32)
        o16 = jnp.ones((16,), jnp.float32)

        def zb(i, c):
            obuf[pl.ds(i * 16, 16)] = z16
            return c
        lax.fori_loop(0, ROWS_PER_SUB // 16, zb, 0)
        pltpu.sync_copy(obuf.at[pl.ds(0, ROWS_PER_SUB)], dacc.at[pl.ds(r0, ROWS_PER_SUB)])

        def ob(i, c):
            obuf[pl.ds(i * 16, 16)] = o16
            return c
        lax.fori_loop(0, (DEG_CHUNK + 15) // 16, ob, 0)

        plsc.subcore_barrier()

        eper = E // (NSUB * NCORE)
        wid = sid * NCORE + cid
        def chunk(i, c):
            b = wid * eper + i * DEG_CHUNK
            pltpu.sync_copy(dst.at[pl.ds(b, DEG_CHUNK)], dstv)
            pltpu.sync_copy(obuf.at[pl.ds(0, DEG_CHUNK)], dacc.at[dstv], add=True)
            return c
        lax.fori_loop(0, eper // DEG_CHUNK, chunk, 0)

        plsc.subcore_barrier()

        pltpu.sync_copy(dacc.at[pl.ds(r0, ROWS_PER_SUB)], obuf.at[pl.ds(0, ROWS_PER_SUB)])

        @pl.when(cid == 0)
        def _():
            pltpu.sync_copy(obuf.at[pl.ds(0, ROWS_PER_SUB)], deg0.at[pl.ds(r0, ROWS_PER_SUB)])

        @pl.when(cid == 1)
        def _():
            pltpu.sync_copy(obuf.at[pl.ds(0, ROWS_PER_SUB)], deg1.at[pl.ds(r0, ROWS_PER_SUB)])

    return pl.kernel(
        body,
        out_type=(jax.ShapeDtypeStruct((NPAD,), jnp.float32),
                  jax.ShapeDtypeStruct((NPAD,), jnp.float32)),
        mesh=_mesh(),
        scratch_types=[
            pltpu.VMEM_SHARED((NPAD,), jnp.float32),
            pltpu.VMEM((DEG_CHUNK,), jnp.int32),
            pltpu.VMEM((((DEG_CHUNK + 15) // 16) * 16,), jnp.float32),
            pltpu.SemaphoreType.DMA,
        ],
    )


def _edge_body(A, B, src3, dst3, out, srcv, dstv, a1, a2, b1_, b2_, sem_a, sem_b):
    cid = lax.axis_index("c")
    sid = lax.axis_index("s")
    wid = sid * NCORE + cid
    eper = E // (NSUB * NCORE)
    base0 = wid * eper

    pltpu.sync_copy(src3.at[wid], srcv)
    pltpu.sync_copy(dst3.at[wid], dstv)

    def fire(i, pa, pb):
        pltpu.async_copy(A.at[srcv.at[i]], pa, sem_a if pa is a1 else sem_b)
        pltpu.async_copy(B.at[dstv.at[i]], pb, sem_a if pa is a1 else sem_b)

    def drain(i, pa, pb):
        sem = sem_a if pa is a1 else sem_b
        pltpu.make_async_copy(A.at[srcv.at[i]], pa, sem).wait()
        pltpu.make_async_copy(B.at[dstv.at[i]], pb, sem).wait()

    def addout(i, pa, pb):
        def addrow(r, c2):
            for cc in range(F // 16):
                s = pl.ds(cc * 16, 16)
                pa[r, s] = pa[r, s] + pb[r, s]
            return c2
        lax.fori_loop(0, EDGE_CHUNK, addrow, 0)
        pltpu.sync_copy(pa, out.at[pl.ds(base0 + i * EDGE_CHUNK, EDGE_CHUNK)])

    fire(0, a1, a2)

    def pairloop(j, c):
        ia = 2 * j
        ib = 2 * j + 1
        fire(ib, b1_, b2_)
        drain(ia, a1, a2)
        addout(ia, a1, a2)
        fire(ib + 1, a1, a2)
        drain(ib, b1_, b2_)
        addout(ib, b1_, b2_)
        return c
    lax.fori_loop(0, (EDGE_NCH - 1) // 2, pairloop, 0)

    last = EDGE_NCH - 1
    drain(last, a1, a2)
    addout(last, a1, a2)


@functools.lru_cache(maxsize=None)
def _make_edge():
    return pl.kernel(
        _edge_body,
        out_type=jax.ShapeDtypeStruct((E, F), jnp.float32),
        mesh=_mesh(),
        scratch_types=[
            pltpu.VMEM((EDGE_NCH, EDGE_CHUNK), jnp.int32),
            pltpu.VMEM((EDGE_NCH, EDGE_CHUNK), jnp.int32),
            pltpu.VMEM((EDGE_CHUNK, F), jnp.float32),
            pltpu.VMEM((EDGE_CHUNK, F), jnp.float32),
            pltpu.VMEM((EDGE_CHUNK, F), jnp.float32),
            pltpu.VMEM((EDGE_CHUNK, F), jnp.float32),
            pltpu.SemaphoreType.DMA,
            pltpu.SemaphoreType.DMA,
        ],
    )


def kernel(h, edge_index, proj_W, proj_b, proj_ln_g, proj_ln_b,
           mp_W, mp_b, mp_ln_g, mp_ln_b, W1, b1, ln_g, ln_b, W2, b2):
    src = edge_index[0]
    dst = edge_index[1]

    deg0, deg1 = _make_deg()(dst)
    h_pad = jnp.pad(h, ((0, NPAD - N), (0, 0)))
    xlo, xhi = _proj(h_pad, proj_W, proj_b, proj_ln_g, proj_ln_b)
    dst3 = dst.reshape(NSUB, AGG_NCH, AGG_CHUNK)
    src3e = src.reshape(NSUB * NCORE, EDGE_NCH, EDGE_CHUNK)
    dst3e = dst.reshape(NSUB * NCORE, EDGE_NCH, EDGE_CHUNK)
    agglo, agghi = _make_agg()(xlo, xhi, src, dst3)
    ylo, yhi = _layer(xlo, xhi, agglo, agghi, deg0, deg1,
                      mp_W[0], mp_b[0].reshape(1, F),
                      mp_ln_g[0].reshape(1, F), mp_ln_b[0].reshape(1, F))
    agglo2, agghi2 = _make_agg()(ylo, yhi, src, dst3)
    A, Bm = _layer_ab(ylo, yhi, agglo2, agghi2, deg0, deg1,
                      mp_W[1], mp_b[1].reshape(1, F),
                      mp_ln_g[1].reshape(1, F), mp_ln_b[1].reshape(1, F),
                      W1, b1.reshape(1, F))
    e_pre = _make_edge()(A, Bm, src3e, dst3e)
    return _final(e_pre, ln_g.reshape(1, F), ln_b.reshape(1, F), W2, b2.reshape(1, CLASSES))


# speedup vs baseline: 5.2196x; 1.0440x over previous
import functools

import jax
import jax.numpy as jnp
from jax import lax
from jax.experimental import pallas as pl
from jax.experimental.pallas import tpu as pltpu
from jax.experimental.pallas import tpu_sc as plsc

N = 10000
E = 160000
F = 256
FH = 128
CLASSES = 2
DEGW = 16

NSUB = 16
NCORE = 2
NPAD = 10240
ROWS_PER_SUB = NPAD // NSUB
AGG_CHUNK = 80
AGG_NCH = E // (NSUB * AGG_CHUNK)
E_SPLITS = ((64000, 80), (96000, 40))

RN = 1024
RE = 2000


def _ln(y, g, b, eps=1e-5):
    m = jnp.mean(y, axis=-1, keepdims=True)
    v = jnp.mean((y - m) ** 2, axis=-1, keepdims=True)
    return (y - m) * lax.rsqrt(v + eps) * g + b


def _proj_body(h_ref, W_ref, b_ref, g_ref, bb_ref, xlo_ref, xhi_ref):
    y0 = jnp.dot(h_ref[:, :FH], W_ref[0], preferred_element_type=jnp.float32) + b_ref[0]
    y1 = jnp.dot(h_ref[:, FH:], W_ref[1], preferred_element_type=jnp.float32) + b_ref[1]
    xlo_ref[...] = jax.nn.relu(_ln(y0, g_ref[0], bb_ref[0]))
    xhi_ref[...] = jax.nn.relu(_ln(y1, g_ref[1], bb_ref[1]))


def _proj(h, proj_W, proj_b, proj_ln_g, proj_ln_b):
    return pl.pallas_call(
        _proj_body,
        grid=(NPAD // RN,),
        in_specs=[
            pl.BlockSpec((RN, F), lambda i: (i, 0)),
            pl.BlockSpec((2, FH, FH), lambda i: (0, 0, 0)),
            pl.BlockSpec((2, FH), lambda i: (0, 0)),
            pl.BlockSpec((2, FH), lambda i: (0, 0)),
            pl.BlockSpec((2, FH), lambda i: (0, 0)),
        ],
        out_specs=[pl.BlockSpec((RN, FH), lambda i: (i, 0))] * 2,
        out_shape=[jax.ShapeDtypeStruct((NPAD, FH), jnp.float32)] * 2,
    )(h, proj_W, proj_b, proj_ln_g, proj_ln_b)


def _layer_common(xlo, xhi, alo, ahi, deg0_ref, deg1_ref, W, b):
    i = pl.program_id(0)
    d = deg0_ref[pl.ds(i * RN, RN)] + deg1_ref[pl.ds(i * RN, RN)]
    d = d.reshape(-1, 1)
    norm = jnp.where(d > 0, 1.0 / d, 0.0)
    y = (jnp.dot(xlo, W[:FH], preferred_element_type=jnp.float32)
         + jnp.dot(xhi, W[FH:F], preferred_element_type=jnp.float32)
         + jnp.dot(alo * norm, W[F:F + FH], preferred_element_type=jnp.float32)
         + jnp.dot(ahi * norm, W[F + FH:], preferred_element_type=jnp.float32)
         + b)
    return y


def _layer_body(xlo_ref, xhi_ref, alo_ref, ahi_ref, deg0_ref, deg1_ref, W_ref, b_ref,
                g_ref, bb_ref, ylo_ref, yhi_ref):
    y = _layer_common(xlo_ref[...], xhi_ref[...], alo_ref[...], ahi_ref[...],
                      deg0_ref, deg1_ref, W_ref[...], b_ref[...])
    y = jax.nn.relu(_ln(y, g_ref[...], bb_ref[...]))
    ylo_ref[...] = y[:, :FH]
    yhi_ref[...] = y[:, FH:]


def _layer(xlo, xhi, alo, ahi, deg0, deg1, W, b, g, bb):
    return pl.pallas_call(
        _layer_body,
        grid=(NPAD // RN,),
        in_specs=[
            pl.BlockSpec((RN, FH), lambda i: (i, 0)),
            pl.BlockSpec((RN, FH), lambda i: (i, 0)),
            pl.BlockSpec((RN, FH), lambda i: (i, 0)),
            pl.BlockSpec((RN, FH), lambda i: (i, 0)),
            pl.BlockSpec((NPAD,), lambda i: (0,)),
            pl.BlockSpec((NPAD,), lambda i: (0,)),
            pl.BlockSpec((2 * F, F), lambda i: (0, 0)),
            pl.BlockSpec((1, F), lambda i: (0, 0)),
            pl.BlockSpec((1, F), lambda i: (0, 0)),
            pl.BlockSpec((1, F), lambda i: (0, 0)),
        ],
        out_specs=[pl.BlockSpec((RN, FH), lambda i: (i, 0))] * 2,
        out_shape=[jax.ShapeDtypeStruct((NPAD, FH), jnp.float32)] * 2,
    )(xlo, xhi, alo, ahi, deg0, deg1, W, b, g, bb)


def _layer_ab_body(xlo_ref, xhi_ref, alo_ref, ahi_ref, deg0_ref, deg1_ref, W_ref, b_ref,
                   g_ref, bb_ref, W1_ref, b1_ref, A_ref, B_ref):
    y = _layer_common(xlo_ref[...], xhi_ref[...], alo_ref[...], ahi_ref[...],
                      deg0_ref, deg1_ref, W_ref[...], b_ref[...])
    y = jax.nn.relu(_ln(y, g_ref[...], bb_ref[...]))
    A_ref[...] = jnp.dot(y, W1_ref[:F], preferred_element_type=jnp.float32)
    B_ref[...] = jnp.dot(y, W1_ref[F:], preferred_element_type=jnp.float32) + b1_ref[...]


def _layer_ab(xlo, xhi, alo, ahi, deg0, deg1, W, b, g, bb, W1, b1):
    return pl.pallas_call(
        _layer_ab_body,
        grid=(NPAD // RN,),
        in_specs=[
            pl.BlockSpec((RN, FH), lambda i: (i, 0)),
            pl.BlockSpec((RN, FH), lambda i: (i, 0)),
            pl.BlockSpec((RN, FH), lambda i: (i, 0)),
            pl.BlockSpec((RN, FH), lambda i: (i, 0)),
            pl.BlockSpec((NPAD,), lambda i: (0,)),
            pl.BlockSpec((NPAD,), lambda i: (0,)),
            pl.BlockSpec((2 * F, F), lambda i: (0, 0)),
            pl.BlockSpec((1, F), lambda i: (0, 0)),
            pl.BlockSpec((1, F), lambda i: (0, 0)),
            pl.BlockSpec((1, F), lambda i: (0, 0)),
            pl.BlockSpec((2 * F, F), lambda i: (0, 0)),
            pl.BlockSpec((1, F), lambda i: (0, 0)),
        ],
        out_specs=[pl.BlockSpec((RN, F), lambda i: (i, 0))] * 2,
        out_shape=[jax.ShapeDtypeStruct((NPAD, F), jnp.float32)] * 2,
    )(xlo, xhi, alo, ahi, deg0, deg1, W, b, g, bb, W1, b1)


def _final_body(e_ref, g_ref, b_ref, W2_ref, b2_ref, o_ref):
    e = jax.nn.relu(_ln(e_ref[...], g_ref[...], b_ref[...]))
    o_ref[...] = jnp.dot(e, W2_ref[...], preferred_element_type=jnp.float32) + b2_ref[...]


def _final(e_pre, g, b, W2, b2, ecount):
    return pl.pallas_call(
        _final_body,
        grid=(ecount // RE,),
        in_specs=[
            pl.BlockSpec((RE, F), lambda i: (i, 0)),
            pl.BlockSpec((1, F), lambda i: (0, 0)),
            pl.BlockSpec((1, F), lambda i: (0, 0)),
            pl.BlockSpec((F, CLASSES), lambda i: (0, 0)),
            pl.BlockSpec((1, CLASSES), lambda i: (0, 0)),
        ],
        out_specs=pl.BlockSpec((RE, CLASSES), lambda i: (i, 0)),
        out_shape=jax.ShapeDtypeStruct((ecount, CLASSES), jnp.float32),
    )(e_pre, g, b, W2, b2)


@functools.lru_cache(maxsize=None)
def _mesh():
    return plsc.VectorSubcoreMesh(core_axis_name="c", subcore_axis_name="s")


@functools.lru_cache(maxsize=None)
def _make_agg():
    def body(xlo, xhi, src1, dst3, agglo, agghi, acc, srcv, dstv,
             rows_a, rows_b, sem_a, sem_b):
        cid = lax.axis_index("c")
        sid = lax.axis_index("s")
        r0 = sid * ROWS_PER_SUB

        z16 = jnp.zeros((16,), jnp.float32)

        def zb(i, c):
            rows_a[i // 8, pl.ds((i % 8) * 16, 16)] = z16
            return c
        lax.fori_loop(0, AGG_CHUNK * 8, zb, 0)

        for j in range(ROWS_PER_SUB // AGG_CHUNK):
            pltpu.sync_copy(rows_a, acc.at[pl.ds(r0 + j * AGG_CHUNK, AGG_CHUNK)])

        eper = E // NSUB
        pltpu.sync_copy(src1.at[pl.ds(sid * eper, eper)], srcv)
        pltpu.sync_copy(dst3.at[sid], dstv)

        plsc.subcore_barrier()

        def sidx(i):
            return srcv.at[pl.ds(i * AGG_CHUNK, AGG_CHUNK)]

        def run(xref):
            pltpu.async_copy(xref.at[sidx(0)], rows_a, sem_a)

            def pair(j, c):
                ia = 2 * j
                ib = 2 * j + 1
                pltpu.async_copy(xref.at[sidx(ib)], rows_b, sem_b)
                pltpu.make_async_copy(xref.at[sidx(ia)], rows_a, sem_a).wait()
                pltpu.sync_copy(rows_a, acc.at[dstv.at[ia]], add=True)
                pltpu.async_copy(xref.at[sidx(ib + 1)], rows_a, sem_a)
                pltpu.make_async_copy(xref.at[sidx(ib)], rows_b, sem_b).wait()
                pltpu.sync_copy(rows_b, acc.at[dstv.at[ib]], add=True)
                return c
            lax.fori_loop(0, (AGG_NCH - 1) // 2, pair, 0)

            last = AGG_NCH - 1
            pltpu.make_async_copy(xref.at[sidx(last)], rows_a, sem_a).wait()
            pltpu.sync_copy(rows_a, acc.at[dstv.at[last]], add=True)

        @pl.when(cid == 0)
        def _():
            run(xlo)

        @pl.when(cid == 1)
        def _():
            run(xhi)

        plsc.subcore_barrier()

        for j in range(ROWS_PER_SUB // AGG_CHUNK):
            sl = pl.ds(r0 + j * AGG_CHUNK, AGG_CHUNK)

            @pl.when(cid == 0)
            def _():
                pltpu.sync_copy(acc.at[sl], rows_a)
                pltpu.sync_copy(rows_a, agglo.at[sl])

            @pl.when(cid == 1)
            def _():
                pltpu.sync_copy(acc.at[sl], rows_a)
                pltpu.sync_copy(rows_a, agghi.at[sl])

    return pl.kernel(
        body,
        out_type=(jax.ShapeDtypeStruct((NPAD, FH), jnp.float32),
                  jax.ShapeDtypeStruct((NPAD, FH), jnp.float32)),
        mesh=_mesh(),
        scratch_types=[
            pltpu.VMEM_SHARED((NPAD, FH), jnp.float32),
            pltpu.VMEM((E // NSUB,), jnp.int32),
            pltpu.VMEM((AGG_NCH, AGG_CHUNK), jnp.int32),
            pltpu.VMEM((AGG_CHUNK, FH), jnp.float32),
            pltpu.VMEM((AGG_CHUNK, FH), jnp.float32),
            pltpu.SemaphoreType.DMA,
            pltpu.SemaphoreType.DMA,
        ],
    )


DEG_CHUNK = 1000


@functools.lru_cache(maxsize=None)
def _make_deg():
    def body(dst, deg0, deg1, dacc, dstv, obuf, sem):
        cid = lax.axis_index("c")
        sid = lax.axis_index("s")
        r0 = sid * ROWS_PER_SUB

        z16 = jnp.zeros((16,), jnp.float32)
        o16 = jnp.ones((16,), jnp.float32)

        def zb(i, c):
            obuf[pl.ds(i * 16, 16)] = z16
            return c
        lax.fori_loop(0, ROWS_PER_SUB // 16, zb, 0)
        pltpu.sync_copy(obuf.at[pl.ds(0, ROWS_PER_SUB)], dacc.at[pl.ds(r0, ROWS_PER_SUB)])

        def ob(i, c):
            obuf[pl.ds(i * 16, 16)] = o16
            return c
        lax.fori_loop(0, (DEG_CHUNK + 15) // 16, ob, 0)

        plsc.subcore_barrier()

        eper = E // (NSUB * NCORE)
        wid = sid * NCORE + cid
        def chunk(i, c):
            b = wid * eper + i * DEG_CHUNK
            pltpu.sync_copy(dst.at[pl.ds(b, DEG_CHUNK)], dstv)
            pltpu.sync_copy(obuf.at[pl.ds(0, DEG_CHUNK)], dacc.at[dstv], add=True)
            return c
        lax.fori_loop(0, eper // DEG_CHUNK, chunk, 0)

        plsc.subcore_barrier()

        pltpu.sync_copy(dacc.at[pl.ds(r0, ROWS_PER_SUB)], obuf.at[pl.ds(0, ROWS_PER_SUB)])

        @pl.when(cid == 0)
        def _():
            pltpu.sync_copy(obuf.at[pl.ds(0, ROWS_PER_SUB)], deg0.at[pl.ds(r0, ROWS_PER_SUB)])

        @pl.when(cid == 1)
        def _():
            pltpu.sync_copy(obuf.at[pl.ds(0, ROWS_PER_SUB)], deg1.at[pl.ds(r0, ROWS_PER_SUB)])

    return pl.kernel(
        body,
        out_type=(jax.ShapeDtypeStruct((NPAD,), jnp.float32),
                  jax.ShapeDtypeStruct((NPAD,), jnp.float32)),
        mesh=_mesh(),
        scratch_types=[
            pltpu.VMEM_SHARED((NPAD,), jnp.float32),
            pltpu.VMEM((DEG_CHUNK,), jnp.int32),
            pltpu.VMEM((((DEG_CHUNK + 15) // 16) * 16,), jnp.float32),
            pltpu.SemaphoreType.DMA,
        ],
    )


@functools.lru_cache(maxsize=None)
def _make_edge(ecount, ch):
    eper = ecount // (NSUB * NCORE)
    nch = eper // ch
    assert nch % 2 == 1 and ch % 8 == 0 and ch <= 128

    def body(A, B, src1, dst1, out, srcv, dstv, a1, a2, b1_, b2_, sem_a, sem_b):
        cid = lax.axis_index("c")
        sid = lax.axis_index("s")
        wid = sid * NCORE + cid
        base0 = wid * eper

        pltpu.sync_copy(src1.at[pl.ds(base0, eper)], srcv)
        pltpu.sync_copy(dst1.at[pl.ds(base0, eper)], dstv)

        def fire(i, bufA, bufB, sem):
            pltpu.async_copy(A.at[srcv.at[pl.ds(i * ch, ch)]], bufA, sem)
            pltpu.async_copy(B.at[dstv.at[pl.ds(i * ch, ch)]], bufB, sem)

        def stage(i, bufA, bufB, sem):
            pltpu.make_async_copy(A.at[srcv.at[pl.ds(i * ch, ch)]], bufA, sem).wait()
            pltpu.make_async_copy(B.at[dstv.at[pl.ds(i * ch, ch)]], bufB, sem).wait()

            def addrow(r, c2):
                for cc in range(F // 16):
                    s = pl.ds(cc * 16, 16)
                    plsc.addupdate(bufA.at[r, s], bufB[r, s])
                return c2
            lax.fori_loop(0, ch, addrow, 0)
            pltpu.sync_copy(bufA, out.at[pl.ds(base0 + i * ch, ch)])

        fire(0, a1, a2, sem_a)

        def pairloop(j, c):
            ia = 2 * j
            ib = 2 * j + 1
            fire(ib, b1_, b2_, sem_b)
            stage(ia, a1, a2, sem_a)
            fire(ib + 1, a1, a2, sem_a)
            stage(ib, b1_, b2_, sem_b)
            return c
        lax.fori_loop(0, (nch - 1) // 2, pairloop, 0)

        stage(nch - 1, a1, a2, sem_a)

    return pl.kernel(
        body,
        out_type=jax.ShapeDtypeStruct((ecount, F), jnp.float32),
        mesh=_mesh(),
        scratch_types=[
            pltpu.VMEM((eper,), jnp.int32),
            pltpu.VMEM((eper,), jnp.int32),
            pltpu.VMEM((ch, F), jnp.float32),
            pltpu.VMEM((ch, F), jnp.float32),
            pltpu.VMEM((ch, F), jnp.float32),
            pltpu.VMEM((ch, F), jnp.float32),
            pltpu.SemaphoreType.DMA,
            pltpu.SemaphoreType.DMA,
        ],
    )


def kernel(h, edge_index, proj_W, proj_b, proj_ln_g, proj_ln_b,
           mp_W, mp_b, mp_ln_g, mp_ln_b, W1, b1, ln_g, ln_b, W2, b2):
    src = edge_index[0]
    dst = edge_index[1]

    deg0, deg1 = _make_deg()(dst)
    h_pad = jnp.pad(h, ((0, NPAD - N), (0, 0)))
    xlo, xhi = _proj(h_pad, proj_W, proj_b, proj_ln_g, proj_ln_b)
    dst3 = dst.reshape(NSUB, AGG_NCH, AGG_CHUNK)

    agglo, agghi = _make_agg()(xlo, xhi, src, dst3)
    ylo, yhi = _layer(xlo, xhi, agglo, agghi, deg0, deg1,
                      mp_W[0], mp_b[0].reshape(1, F),
                      mp_ln_g[0].reshape(1, F), mp_ln_b[0].reshape(1, F))
    agglo2, agghi2 = _make_agg()(ylo, yhi, src, dst3)
    A, Bm = _layer_ab(ylo, yhi, agglo2, agghi2, deg0, deg1,
                      mp_W[1], mp_b[1].reshape(1, F),
                      mp_ln_g[1].reshape(1, F), mp_ln_b[1].reshape(1, F),
                      W1, b1.reshape(1, F))
    outs = []
    off = 0
    for ecount, ch in E_SPLITS:
        s1 = lax.slice_in_dim(src, off, off + ecount)
        d1 = lax.slice_in_dim(dst, off, off + ecount)
        e_pre = _make_edge(ecount, ch)(A, Bm, s1, d1)
        outs.append(_final(e_pre, ln_g.reshape(1, F), ln_b.reshape(1, F),
                           W2, b2.reshape(1, CLASSES), ecount))
        off += ecount
    return jnp.concatenate(outs, axis=0)
